# baseline (speedup 1.0000x reference)
import jax
import jax.numpy as jnp
from jax.experimental import pallas as pl
from jax.experimental.pallas import tpu as pltpu


_NSPLIT = 5
_BM_SUB = 40


def _fused_body(*refs):
    adj_refs = refs[:_NSPLIT]
    x_ref, w_ref, o_ref, sup_ref = refs[_NSPLIT:]

    @pl.when(pl.program_id(0) == 0)
    def _():
        sup_ref[...] = jnp.dot(
            x_ref[...].astype(jnp.bfloat16),
            w_ref[...].astype(jnp.bfloat16),
            preferred_element_type=jnp.float32).astype(jnp.bfloat16)

    for j, a_ref in enumerate(adj_refs):
        o_ref[j * _BM_SUB:(j + 1) * _BM_SUB, :] = jnp.dot(
            a_ref[...].astype(jnp.bfloat16), sup_ref[...],
            preferred_element_type=jnp.float32)


def _largest_divisor(n, target, step=8):
    best = step
    d = step
    while d <= target:
        if n % d == 0:
            best = d
        d += step
    return best


@jax.jit
def kernel(input, adj, weight):
    n, d_in = input.shape
    d_out = weight.shape[1]

    bm = _NSPLIT * _BM_SUB
    adj_specs = [
        pl.BlockSpec((_BM_SUB, n), lambda i, j=j: (i * _NSPLIT + j, 0))
        for j in range(_NSPLIT)
    ]
    out = pl.pallas_call(
        _fused_body,
        grid=(n // bm,),
        in_specs=adj_specs + [
            pl.BlockSpec((n, d_in), lambda i: (0, 0)),
            pl.BlockSpec((d_in, d_out), lambda i: (0, 0)),
        ],
        out_specs=pl.BlockSpec((bm, d_out), lambda i: (i, 0)),
        out_shape=jax.ShapeDtypeStruct((n, d_out), jnp.float32),
        scratch_shapes=[pltpu.VMEM((n, d_out), jnp.bfloat16)],
        compiler_params=pltpu.CompilerParams(
            dimension_semantics=("arbitrary",)),
    )(*([adj] * _NSPLIT), input, weight)
    return out


# speedup vs baseline: 1.8324x; 1.8324x over previous
import jax
import jax.numpy as jnp
from jax.experimental import pallas as pl
from jax.experimental.pallas import tpu as pltpu


_BM = 400


def _fused_body(adj_ref, x_ref, w_ref, o_ref, sup_ref):
    @pl.when(pl.program_id(0) == 0)
    def _():
        sup_ref[...] = jnp.dot(
            x_ref[...].astype(jnp.bfloat16),
            w_ref[...].astype(jnp.bfloat16),
            preferred_element_type=jnp.float32).astype(jnp.bfloat16)

    o_ref[...] = jnp.dot(adj_ref[...].astype(jnp.bfloat16), sup_ref[...],
                         preferred_element_type=jnp.float32)


def _largest_divisor(n, target, step=8):
    best = step
    d = step
    while d <= target:
        if n % d == 0:
            best = d
        d += step
    return best


@jax.jit
def kernel(input, adj, weight):
    n, d_in = input.shape
    d_out = weight.shape[1]

    bm = _BM
    out = pl.pallas_call(
        _fused_body,
        grid=(n // bm,),
        in_specs=[
            pl.BlockSpec((bm, n), lambda i: (i, 0)),
            pl.BlockSpec((n, d_in), lambda i: (0, 0)),
            pl.BlockSpec((d_in, d_out), lambda i: (0, 0)),
        ],
        out_specs=pl.BlockSpec((bm, d_out), lambda i: (i, 0)),
        out_shape=jax.ShapeDtypeStruct((n, d_out), jnp.float32),
        scratch_shapes=[pltpu.VMEM((n, d_out), jnp.bfloat16)],
        compiler_params=pltpu.CompilerParams(
            dimension_semantics=("arbitrary",)),
    )(adj, input, weight)
    return out


# speedup vs baseline: 1.8570x; 1.0134x over previous
import jax
import jax.numpy as jnp
from jax.experimental import pallas as pl
from jax.experimental.pallas import tpu as pltpu


_NSPLIT = 2
_BM_SUB = 200


def _fused_body(*refs):
    adj_refs = refs[:_NSPLIT]
    x_ref, w_ref, o_ref, sup_ref = refs[_NSPLIT:]

    @pl.when(pl.program_id(0) == 0)
    def _():
        sup_ref[...] = jnp.dot(
            x_ref[...].astype(jnp.bfloat16),
            w_ref[...].astype(jnp.bfloat16),
            preferred_element_type=jnp.float32).astype(jnp.bfloat16)

    for j, a_ref in enumerate(adj_refs):
        o_ref[j * _BM_SUB:(j + 1) * _BM_SUB, :] = jnp.dot(
            a_ref[...].astype(jnp.bfloat16), sup_ref[...],
            preferred_element_type=jnp.float32)


def _largest_divisor(n, target, step=8):
    best = step
    d = step
    while d <= target:
        if n % d == 0:
            best = d
        d += step
    return best


@jax.jit
def kernel(input, adj, weight):
    n, d_in = input.shape
    d_out = weight.shape[1]

    bm = _NSPLIT * _BM_SUB
    adj_specs = [
        pl.BlockSpec((_BM_SUB, n), lambda i, j=j: (i * _NSPLIT + j, 0))
        for j in range(_NSPLIT)
    ]
    out = pl.pallas_call(
        _fused_body,
        grid=(n // bm,),
        in_specs=adj_specs + [
            pl.BlockSpec((n, d_in), lambda i: (0, 0)),
            pl.BlockSpec((d_in, d_out), lambda i: (0, 0)),
        ],
        out_specs=pl.BlockSpec((bm, d_out), lambda i: (i, 0)),
        out_shape=jax.ShapeDtypeStruct((n, d_out), jnp.float32),
        scratch_shapes=[pltpu.VMEM((n, d_out), jnp.bfloat16)],
        compiler_params=pltpu.CompilerParams(
            dimension_semantics=("arbitrary",)),
    )(*([adj] * _NSPLIT), input, weight)
    return out
